# Initial kernel scaffold; baseline (speedup 1.0000x reference)
import jax
import jax.numpy as jnp
from jax.experimental import pallas as pl


def kernel(audio, x, edge_index, params):
    raise NotImplementedError("write your pallas kernel here")


# speedup vs baseline: 5.2584x; 5.2584x over previous
import functools

import jax
import jax.numpy as jnp
from jax import lax
from jax.experimental import pallas as pl
from jax.experimental.pallas import tpu as pltpu

_N_TILE = 400


def _ln(h, g, b):
    mu = jnp.mean(h, axis=-1, keepdims=True)
    var = jnp.mean((h - mu) ** 2, axis=-1, keepdims=True)
    return (h - mu) * lax.rsqrt(var + 1e-5) * g + b


def _k1(x_ref, w_ref, dinv_ref, o_ref):
    h = jnp.dot(x_ref[...], w_ref[...], preferred_element_type=jnp.float32)
    o_ref[...] = h * dinv_ref[...][:, 0:1]


def _k2(agg_ref, dinv_ref, pk_ref, w2_ref, o_ref):
    dv = dinv_ref[...][:, 0:1]
    h = agg_ref[...] * dv + pk_ref[0:1, :64]
    h = jnp.maximum(h, 0.0)
    h = _ln(h, pk_ref[1:2, :64], pk_ref[2:3, :64])
    o_ref[...] = jnp.dot(h, w2_ref[...], preferred_element_type=jnp.float32) * dv


def _k3(agg_ref, dinv_ref, pk_ref, wg_ref, a_ref, hg_ref, at_ref):
    dv = dinv_ref[...][:, 0:1]
    h = agg_ref[...] * dv + pk_ref[0:1, :128]
    h = jnp.maximum(h, 0.0)
    h = _ln(h, pk_ref[1:2, :128], pk_ref[2:3, :128])
    hg = jnp.dot(h, wg_ref[...], preferred_element_type=jnp.float32)
    hg_ref[...] = hg
    alr = jnp.dot(hg, a_ref[...], preferred_element_type=jnp.float32)
    at_ref[...] = jnp.concatenate(
        [alr, jnp.zeros((alr.shape[0], 120), jnp.float32)], axis=1)


def _k4(num_ref, den_ref, pk_ref, ga1_ref, ga2_ref, x3_ref, gl_ref):
    den = den_ref[...]
    t = den.shape[0]
    den_rep = jnp.concatenate(
        [jnp.broadcast_to(den[:, i:i + 1], (t, 64)) for i in range(4)], axis=1)
    gat = num_ref[...] / (den_rep + 1e-16)
    x3 = _ln(gat + pk_ref[0:1, :256], pk_ref[1:2, :256], pk_ref[2:3, :256])
    x3_ref[...] = x3
    g1 = jnp.maximum(
        jnp.dot(x3, ga1_ref[...], preferred_element_type=jnp.float32)
        + pk_ref[3:4, :256], 0.0)
    gl = jnp.dot(g1, ga2_ref[...], preferred_element_type=jnp.float32)
    gl_ref[...] = gl + pk_ref[4:5, :128]


def _k5(x3_ref, fr_ref, dfr_ref, va1_ref, va2_ref, cm1_ref, cm2_ref, cm3_ref,
        vp1_ref, vp2_ref, xp1_ref, xp2_ref, yp1_ref, yp2_ref, zp1_ref,
        zp2_ref, bb_ref, cb_ref, o_ref):
    x3 = x3_ref[...]
    t = x3.shape[0]
    vwg = jnp.concatenate([x3, jnp.broadcast_to(fr_ref[0:1, 256:], (t, 256))],
                          axis=1)
    aw1 = jnp.maximum(
        jnp.dot(vwg, va1_ref[...], preferred_element_type=jnp.float32)
        + bb_ref[0:1, :128], 0.0)
    awl = jnp.dot(aw1, va2_ref[...], preferred_element_type=jnp.float32)
    aw = jax.nn.sigmoid(awl[:, 0:1] + cb_ref[0:1, 4:5])
    av = vwg * aw
    h = jnp.maximum(
        jnp.dot(av, cm1_ref[...], preferred_element_type=jnp.float32)
        + bb_ref[1:2, :128], 0.0)
    h = jnp.maximum(
        jnp.dot(h, cm2_ref[...], preferred_element_type=jnp.float32)
        + bb_ref[2:3, :64], 0.0)
    contact = jnp.dot(h, cm3_ref[...],
                      preferred_element_type=jnp.float32)[:, 0:1]
    vg = jnp.concatenate(
        [x3, jnp.broadcast_to(dfr_ref[0:1, 256:], (t, 256))], axis=1)
    vp = jnp.maximum(
        jnp.dot(vg, vp1_ref[...], preferred_element_type=jnp.float32)
        + bb_ref[3:4, :256], 0.0)
    vp = jnp.maximum(
        jnp.dot(vp, vp2_ref[...], preferred_element_type=jnp.float32)
        + bb_ref[4:5, :256], 0.0)

    def head(w1, b_row, w2):
        hh = jnp.maximum(
            jnp.dot(vp, w1[...], preferred_element_type=jnp.float32)
            + b_row, 0.0)
        return jnp.dot(hh, w2[...], preferred_element_type=jnp.float32)[:, 0:1]

    dx = head(xp1_ref, bb_ref[5:6, :64], xp2_ref)
    dy = head(yp1_ref, bb_ref[6:7, :64], yp2_ref)
    dz = head(zp1_ref, bb_ref[7:8, :64], zp2_ref)
    out = jnp.concatenate(
        [contact, dx, dy, dz,
         jnp.zeros((t, 124), jnp.float32)], axis=1)
    o_ref[...] = out + cb_ref[0:1, :] * jnp.concatenate(
        [jnp.ones((t, 4), jnp.float32), jnp.zeros((t, 124), jnp.float32)],
        axis=1)


def _rowspec(shape):
    nd = len(shape)
    return pl.BlockSpec((_N_TILE,) + shape[1:],
                        lambda i: (i,) + (0,) * (nd - 1))


def _cspec(shape):
    nd = len(shape)
    return pl.BlockSpec(shape, lambda i: (0,) * nd)


def _grid_call(body, ins, in_specs, out_shapes, out_specs):
    return pl.pallas_call(
        body,
        grid=(10000 // _N_TILE,),
        in_specs=in_specs,
        out_specs=out_specs,
        out_shape=out_shapes,
    )(*ins)


def kernel(audio, x, edge_index, params):
    p = params
    n = x.shape[0]
    f32 = jnp.float32

    def conv(a, w, b):
        y = lax.conv_general_dilated(a, w, (1, 1), ((1, 1), (1, 1)),
                                     dimension_numbers=('NCHW', 'OIHW', 'NCHW'))
        return y + b[None, :, None, None]

    def bn(a, g, bt, m, v):
        return (a - m[None, :, None, None]) * lax.rsqrt(
            v[None, :, None, None] + 1e-5) * g[None, :, None, None] \
            + bt[None, :, None, None]

    def lrelu(a, s=0.2):
        return jnp.where(a > 0, a, s * a)

    def mp(a, kh, kw):
        return lax.reduce_window(a, -jnp.inf, lax.max, (1, 1, kh, kw),
                                 (1, 1, kh, kw), 'VALID')

    a = mp(lrelu(bn(conv(audio, p['c1_w'], p['c1_b']), p['bn1_g'], p['bn1_b'],
                    p['bn1_m'], p['bn1_v'])), 2, 1)
    a = mp(lrelu(bn(conv(a, p['c2_w'], p['c2_b']), p['bn2_g'], p['bn2_b'],
                    p['bn2_m'], p['bn2_v'])), 2, 1)
    a = mp(lrelu(bn(conv(a, p['c3_w'], p['c3_b']), p['bn3_g'], p['bn3_b'],
                    p['bn3_m'], p['bn3_v'])), 2, 2)
    B, C, H, W = a.shape
    a = a.reshape(B, C, 4, H // 4, 2, W // 2).mean(axis=(3, 5)).reshape(B, -1)
    a = lrelu(a @ p['fc_w'] + p['fc_b'])
    att = jax.nn.sigmoid(a @ p['att_w'] + p['att_b'])
    audio_feat = a * att

    loops = jnp.arange(n, dtype=edge_index.dtype)
    src = jnp.concatenate([edge_index[0], loops])
    dst = jnp.concatenate([edge_index[1], loops])
    e_tot = src.shape[0]

    deg = jax.ops.segment_sum(jnp.ones((e_tot,), f32), dst, n)
    dinv = jnp.where(deg > 0, lax.rsqrt(deg), 0.0)
    dinv2d = jnp.broadcast_to(dinv[:, None], (n, 128))

    def segsum(rows, d):
        return jax.ops.segment_sum(rows, d, n)

    x8 = jnp.concatenate([x, jnp.zeros((n, 5), f32)], axis=1)
    w1p = jnp.concatenate([p['g1_w'], jnp.zeros((5, 64), f32)], axis=0)
    h1s = _grid_call(_k1, (x8, w1p, dinv2d),
                     [_rowspec((n, 8)), _cspec((8, 64)), _rowspec((n, 128))],
                     jax.ShapeDtypeStruct((n, 64), f32), _rowspec((n, 64)))
    agg1 = segsum(h1s[src], dst)

    pk2 = jnp.stack([p['g1_b'], p['ln1_g'], p['ln1_b']], axis=0)
    pk2 = jnp.concatenate([pk2, jnp.zeros((3, 64), f32)], axis=1)
    h2s = _grid_call(_k2, (agg1, dinv2d, pk2, p['g2_w']),
                     [_rowspec((n, 64)), _rowspec((n, 128)), _cspec((3, 128)),
                      _cspec((64, 128))],
                     jax.ShapeDtypeStruct((n, 128), f32), _rowspec((n, 128)))
    agg2 = segsum(h2s[src], dst)

    pk3 = jnp.stack([p['g2_b'], p['ln2_g'], p['ln2_b']], axis=0)
    amat = jnp.zeros((256, 8), f32)
    for hd in range(4):
        amat = amat.at[hd * 64:(hd + 1) * 64, hd].set(p['gat_as'][hd])
        amat = amat.at[hd * 64:(hd + 1) * 64, 4 + hd].set(p['gat_ad'][hd])
    hg, at = _grid_call(
        _k3, (agg2, dinv2d, pk3, p['gat_w'], amat),
        [_rowspec((n, 128)), _rowspec((n, 128)), _cspec((3, 128)),
         _cspec((128, 256)), _cspec((256, 8))],
        (jax.ShapeDtypeStruct((n, 256), f32),
         jax.ShapeDtypeStruct((n, 128), f32)),
        (_rowspec((n, 256)), _rowspec((n, 128))))

    e = lrelu(at[src, 0:4] + at[dst, 4:8], 0.2)
    ge = jnp.exp(e)
    num = segsum(hg[src] * jnp.repeat(ge, 64, axis=1), dst)
    den = jnp.concatenate([segsum(ge, dst), jnp.zeros((n, 124), f32)], axis=1)

    pk4 = jnp.stack([p['gat_b'], p['ln3_g'], p['ln3_b'],
                     p['ga1_b']], axis=0)
    pk4 = jnp.concatenate(
        [pk4, jnp.concatenate([jnp.broadcast_to(p['ga2_b'][None, :], (1, 1)),
                               jnp.zeros((1, 255), f32)], axis=1)], axis=0)
    ga2p = jnp.concatenate([p['ga2_w'], jnp.zeros((256, 127), f32)], axis=1)
    x3, gl = _grid_call(
        _k4, (num, den, pk4, p['ga1_w'], ga2p),
        [_rowspec((n, 256)), _rowspec((n, 128)), _cspec((5, 256)),
         _cspec((256, 256)), _cspec((256, 128))],
        (jax.ShapeDtypeStruct((n, 256), f32),
         jax.ShapeDtypeStruct((n, 128), f32)),
        (_rowspec((n, 256)), _rowspec((n, 128))))

    w = jax.nn.softmax(gl[:, 0], axis=0)[:, None]
    mesh_global = jnp.maximum((w * x3).sum(0, keepdims=True) @ p['mm_w']
                              + p['mm_b'], 0.0)
    gf = jnp.concatenate([audio_feat, mesh_global], axis=1)
    f = _ln(jnp.maximum(gf @ p['cf1_w'] + p['cf1_b'], 0.0),
            p['cln1_g'], p['cln1_b'])
    f = _ln(jnp.maximum(f @ p['cf2_w'] + p['cf2_b'], 0.0),
            p['cln2_g'], p['cln2_b']) + f
    df = lrelu(gf @ p['df1_w'] + p['df1_b'])
    df = lrelu(df @ p['df2_w'] + p['df2_b'])

    frow = jnp.concatenate([jnp.zeros((1, 256), f32), f], axis=1)
    frow = jnp.concatenate([frow, jnp.zeros((7, 512), f32)], axis=0)
    dfrow = jnp.concatenate([jnp.zeros((1, 256), f32), df], axis=1)
    dfrow = jnp.concatenate([dfrow, jnp.zeros((7, 512), f32)], axis=0)

    def padrow(v, width):
        return jnp.concatenate([v, jnp.zeros((width - v.shape[0],), f32)])

    bb = jnp.stack([padrow(p['va1_b'], 256), padrow(p['cm1_b'], 256),
                    padrow(p['cm2_b'], 256), padrow(p['vp1_b'], 256),
                    padrow(p['vp2_b'], 256), padrow(p['xp1_b'], 256),
                    padrow(p['yp1_b'], 256), padrow(p['zp1_b'], 256)], axis=0)
    cb = jnp.zeros((1, 128), f32)
    cb = cb.at[0, 0].set(p['cm3_b'][0]).at[0, 1].set(p['xp2_b'][0])
    cb = cb.at[0, 2].set(p['yp2_b'][0]).at[0, 3].set(p['zp2_b'][0])
    cb = cb.at[0, 4].set(p['va2_b'][0])

    def padcol(m, width=128):
        return jnp.concatenate(
            [m, jnp.zeros((m.shape[0], width - m.shape[1]), f32)], axis=1)

    out = _grid_call(
        _k5,
        (x3, frow, dfrow, p['va1_w'], padcol(p['va2_w']), p['cm1_w'],
         p['cm2_w'], padcol(p['cm3_w']), p['vp1_w'], p['vp2_w'], p['xp1_w'],
         padcol(p['xp2_w']), p['yp1_w'], padcol(p['yp2_w']), p['zp1_w'],
         padcol(p['zp2_w']), bb, cb),
        [_rowspec((n, 256)), _cspec((8, 512)), _cspec((8, 512)),
         _cspec((512, 128)), _cspec((128, 128)), _cspec((512, 128)),
         _cspec((128, 64)), _cspec((64, 128)), _cspec((512, 256)),
         _cspec((256, 256)), _cspec((256, 64)), _cspec((64, 128)),
         _cspec((256, 64)), _cspec((64, 128)), _cspec((256, 64)),
         _cspec((64, 128)), _cspec((8, 256)), _cspec((1, 128))],
        jax.ShapeDtypeStruct((n, 128), f32), _rowspec((n, 128)))

    contact = out[:, 0]
    disp = out[:, 1:4]
    return contact, disp
